# baseline (speedup 1.0000x reference)
import functools

import jax
import jax.numpy as jnp
from jax import lax
from jax.experimental import pallas as pl
from jax.experimental.pallas import tpu as pltpu
from jax.experimental.pallas import tpu_sc as plsc

BATCH = 4
SEQ = 4096
D_MODEL = 768

NUM_CORES = 2
NUM_SUBCORES = 16
NW = NUM_CORES * NUM_SUBCORES
ROWS_PER_W = SEQ // NW
CH = 32
CHW = CH * D_MODEL
N_CH = ROWS_PER_W // CH
LANES = 16


def _sc_body(x_hbm, pe_hbm, out_hbm, pe_v, x_v):
    wid = lax.axis_index("s") * NUM_CORES + lax.axis_index("c")
    base = wid * (ROWS_PER_W * D_MODEL)
    for t in range(N_CH):
        off = base + t * CHW
        pltpu.sync_copy(pe_hbm.at[pl.ds(off, CHW)], pe_v)
        for b in range(BATCH):
            xoff = b * (SEQ * D_MODEL) + off
            pltpu.sync_copy(x_hbm.at[pl.ds(xoff, CHW)], x_v)

            @plsc.parallel_loop(0, CHW // LANES, unroll=8)
            def _(i):
                o = i * LANES
                x_v[pl.ds(o, LANES)] = x_v[pl.ds(o, LANES)] + pe_v[pl.ds(o, LANES)]

            pltpu.sync_copy(x_v, out_hbm.at[pl.ds(xoff, CHW)])


@jax.jit
def _sc_add(xf, pef):
    run = functools.partial(
        pl.kernel,
        out_type=jax.ShapeDtypeStruct((BATCH * SEQ * D_MODEL,), jnp.float32),
        mesh=plsc.VectorSubcoreMesh(core_axis_name="c", subcore_axis_name="s"),
        scratch_types=[
            pltpu.VMEM((CHW,), jnp.float32),
            pltpu.VMEM((CHW,), jnp.float32),
        ],
    )(_sc_body)
    return run(xf, pef)


def kernel(x, pos_embed):
    out = _sc_add(x.reshape(-1), pos_embed.reshape(-1))
    return out.reshape(BATCH, SEQ, D_MODEL)


# speedup vs baseline: 1.1769x; 1.1769x over previous
import functools

import jax
import jax.numpy as jnp
from jax import lax
from jax.experimental import pallas as pl
from jax.experimental.pallas import tpu as pltpu
from jax.experimental.pallas import tpu_sc as plsc

BATCH = 4
SEQ = 4096
D_MODEL = 768

NUM_CORES = 2
NUM_SUBCORES = 16
NW = NUM_CORES * NUM_SUBCORES
ROWS_PER_W = SEQ // NW
CH = 32
CHW = CH * D_MODEL
N_CH = ROWS_PER_W // CH
LANES = 16


N_ITEMS = N_CH * BATCH


def _sc_body(x_hbm, pe_hbm, out_hbm, pe0, pe1, x0, x1,
             sp0, sp1, sl0, sl1, ss0, ss1):
    pe_v, x_v = (pe0, pe1), (x0, x1)
    sp, sl, ss = (sp0, sp1), (sl0, sl1), (ss0, ss1)
    wid = lax.axis_index("s") * NUM_CORES + lax.axis_index("c")
    base = wid * (ROWS_PER_W * D_MODEL)

    def x_off(i):
        t, b = divmod(i, BATCH)
        return b * (SEQ * D_MODEL) + base + t * CHW

    pe_desc = [None, None]
    load_desc = [None, None]
    store_desc = [None, None]

    pe_desc[0] = pltpu.async_copy(pe_hbm.at[pl.ds(base, CHW)], pe0, sp0)
    load_desc[0] = pltpu.async_copy(x_hbm.at[pl.ds(x_off(0), CHW)], x0, sl0)

    for i in range(N_ITEMS):
        j = i % 2
        t = i // BATCH
        nxt = i + 1
        if nxt < N_ITEMS:
            if store_desc[1 - j] is not None:
                store_desc[1 - j].wait()
            load_desc[1 - j] = pltpu.async_copy(
                x_hbm.at[pl.ds(x_off(nxt), CHW)], x_v[1 - j], sl[1 - j])
            if nxt % BATCH == 0:
                nt = nxt // BATCH
                pe_desc[nt % 2] = pltpu.async_copy(
                    pe_hbm.at[pl.ds(base + nt * CHW, CHW)], pe_v[nt % 2],
                    sp[nt % 2])
        load_desc[j].wait()
        if i % BATCH == 0:
            pe_desc[t % 2].wait()
        xb, pb = x_v[j], pe_v[t % 2]

        @plsc.parallel_loop(0, CHW // LANES, unroll=8)
        def _(k):
            o = k * LANES
            xb[pl.ds(o, LANES)] = xb[pl.ds(o, LANES)] + pb[pl.ds(o, LANES)]

        store_desc[j] = pltpu.async_copy(xb, out_hbm.at[pl.ds(x_off(i), CHW)],
                                         ss[j])
    store_desc[0].wait()
    store_desc[1].wait()


@jax.jit
def _sc_add(xf, pef):
    run = functools.partial(
        pl.kernel,
        out_type=jax.ShapeDtypeStruct((BATCH * SEQ * D_MODEL,), jnp.float32),
        mesh=plsc.VectorSubcoreMesh(core_axis_name="c", subcore_axis_name="s"),
        scratch_types=[
            pltpu.VMEM((CHW,), jnp.float32),
            pltpu.VMEM((CHW,), jnp.float32),
            pltpu.VMEM((CHW,), jnp.float32),
            pltpu.VMEM((CHW,), jnp.float32),
            pltpu.SemaphoreType.DMA,
            pltpu.SemaphoreType.DMA,
            pltpu.SemaphoreType.DMA,
            pltpu.SemaphoreType.DMA,
            pltpu.SemaphoreType.DMA,
            pltpu.SemaphoreType.DMA,
        ],
    )(_sc_body)
    return run(xf, pef)


def kernel(x, pos_embed):
    out = _sc_add(x.reshape(-1), pos_embed.reshape(-1))
    return out.reshape(BATCH, SEQ, D_MODEL)


# speedup vs baseline: 1.2302x; 1.0453x over previous
import functools

import jax
import jax.numpy as jnp
from jax import lax
from jax.experimental import pallas as pl
from jax.experimental.pallas import tpu as pltpu
from jax.experimental.pallas import tpu_sc as plsc

BATCH = 4
SEQ = 4096
D_MODEL = 768

NUM_CORES = 2
NUM_SUBCORES = 16
NW = NUM_CORES * NUM_SUBCORES
ROWS_PER_W = SEQ // NW
CH = 16
CHW = CH * D_MODEL
N_CH = ROWS_PER_W // CH
LANES = 16
NBUF = 6
PF = NBUF - 2

N_ITEMS = N_CH * BATCH


def _sc_body(x_hbm, pe_hbm, out_hbm, *refs):
    x_v = refs[0:NBUF]
    pe_v = refs[NBUF:NBUF + 2]
    sl = refs[NBUF + 2:2 * NBUF + 2]
    ss = refs[2 * NBUF + 2:3 * NBUF + 2]
    sp = refs[3 * NBUF + 2:3 * NBUF + 4]
    wid = lax.axis_index("s") * NUM_CORES + lax.axis_index("c")
    base = wid * (ROWS_PER_W * D_MODEL)

    def x_off(i):
        t, b = divmod(i, BATCH)
        return b * (SEQ * D_MODEL) + base + t * CHW

    pe_desc = [None, None]
    load_desc = [None] * NBUF
    store_desc = [None] * NBUF

    def fire_load(k):
        m = k % NBUF
        if store_desc[m] is not None:
            store_desc[m].wait()
        load_desc[m] = pltpu.async_copy(
            x_hbm.at[pl.ds(x_off(k), CHW)], x_v[m], sl[m])
        if k % BATCH == 0:
            t = k // BATCH
            pe_desc[t % 2] = pltpu.async_copy(
                pe_hbm.at[pl.ds(base + t * CHW, CHW)], pe_v[t % 2], sp[t % 2])

    for k in range(PF):
        fire_load(k)
    for i in range(N_ITEMS):
        if i + PF < N_ITEMS:
            fire_load(i + PF)
        m = i % NBUF
        t = i // BATCH
        load_desc[m].wait()
        if i % BATCH == 0:
            pe_desc[t % 2].wait()
        xb, pb = x_v[m], pe_v[t % 2]

        @plsc.parallel_loop(0, CHW // LANES, unroll=8)
        def _(k):
            o = k * LANES
            xb[pl.ds(o, LANES)] = xb[pl.ds(o, LANES)] + pb[pl.ds(o, LANES)]

        store_desc[m] = pltpu.async_copy(xb, out_hbm.at[pl.ds(x_off(i), CHW)],
                                         ss[m])
    for m in range(NBUF):
        store_desc[m].wait()


@jax.jit
def _sc_add(xf, pef):
    run = functools.partial(
        pl.kernel,
        out_type=jax.ShapeDtypeStruct((BATCH * SEQ * D_MODEL,), jnp.float32),
        mesh=plsc.VectorSubcoreMesh(core_axis_name="c", subcore_axis_name="s"),
        scratch_types=(
            [pltpu.VMEM((CHW,), jnp.float32)] * NBUF
            + [pltpu.VMEM((CHW,), jnp.float32)] * 2
            + [pltpu.SemaphoreType.DMA] * NBUF
            + [pltpu.SemaphoreType.DMA] * NBUF
            + [pltpu.SemaphoreType.DMA] * 2
        ),
    )(_sc_body)
    return run(xf, pef)


def kernel(x, pos_embed):
    out = _sc_add(x.reshape(-1), pos_embed.reshape(-1))
    return out.reshape(BATCH, SEQ, D_MODEL)


# speedup vs baseline: 2.6661x; 2.1671x over previous
import functools

import jax
import jax.numpy as jnp
from jax import lax
from jax.experimental import pallas as pl
from jax.experimental.pallas import tpu as pltpu
from jax.experimental.pallas import tpu_sc as plsc

BATCH = 4
SEQ = 4096
D_MODEL = 768

NUM_CORES = 2
NUM_SUBCORES = 16
NW = NUM_CORES * NUM_SUBCORES
ROWS_PER_W = SEQ // NW
CH = 16
N_CH = ROWS_PER_W // CH
LANES = 16
VPR = D_MODEL // LANES
NBUF = 6
PF = NBUF - 2

N_ITEMS = N_CH * BATCH


def _sc_body(x_hbm, pe_hbm, out_hbm, *refs):
    x_v = refs[0:NBUF]
    pe_v = refs[NBUF:NBUF + 2]
    sl = refs[NBUF + 2:2 * NBUF + 2]
    ss = refs[2 * NBUF + 2:3 * NBUF + 2]
    sp = refs[3 * NBUF + 2:3 * NBUF + 4]
    wid = lax.axis_index("s") * NUM_CORES + lax.axis_index("c")
    row0 = wid * ROWS_PER_W

    pe_desc = [None, None]
    load_desc = [None] * NBUF
    store_desc = [None] * NBUF

    def fire_load(k):
        m = k % NBUF
        t, b = divmod(k, BATCH)
        if store_desc[m] is not None:
            store_desc[m].wait()
        load_desc[m] = pltpu.async_copy(
            x_hbm.at[b, pl.ds(row0 + t * CH, CH), :], x_v[m], sl[m])
        if b == 0:
            pe_desc[t % 2] = pltpu.async_copy(
                pe_hbm.at[pl.ds(row0 + t * CH, CH), :], pe_v[t % 2], sp[t % 2])

    for k in range(PF):
        fire_load(k)
    for i in range(N_ITEMS):
        if i + PF < N_ITEMS:
            fire_load(i + PF)
        m = i % NBUF
        t, b = divmod(i, BATCH)
        load_desc[m].wait()
        if b == 0:
            pe_desc[t % 2].wait()
        xb, pb = x_v[m], pe_v[t % 2]

        @plsc.parallel_loop(0, CH * VPR, unroll=8)
        def _(k):
            r = k // VPR
            o = (k % VPR) * LANES
            xb[r, pl.ds(o, LANES)] = (xb[r, pl.ds(o, LANES)]
                                      + pb[r, pl.ds(o, LANES)])

        store_desc[m] = pltpu.async_copy(
            xb, out_hbm.at[b, pl.ds(row0 + t * CH, CH), :], ss[m])
    for m in range(NBUF):
        store_desc[m].wait()


@jax.jit
def _sc_add(x, pe):
    run = functools.partial(
        pl.kernel,
        out_type=jax.ShapeDtypeStruct((BATCH, SEQ, D_MODEL), jnp.float32),
        mesh=plsc.VectorSubcoreMesh(core_axis_name="c", subcore_axis_name="s"),
        scratch_types=(
            [pltpu.VMEM((CH, D_MODEL), jnp.float32)] * NBUF
            + [pltpu.VMEM((CH, D_MODEL), jnp.float32)] * 2
            + [pltpu.SemaphoreType.DMA] * NBUF
            + [pltpu.SemaphoreType.DMA] * NBUF
            + [pltpu.SemaphoreType.DMA] * 2
        ),
    )(_sc_body)
    return run(x, pe)


def kernel(x, pos_embed):
    return _sc_add(x, pos_embed)
